# baseline (speedup 1.0000x reference)
import functools

import jax
import jax.numpy as jnp
from jax import lax
from jax.experimental import pallas as pl
from jax.experimental.pallas import tpu as pltpu
from jax.experimental.pallas import tpu_sc as plsc

N = 10000
E = 320000
D = 128
NC, NS = 2, 16
NW = NC * NS
CHUNK = 128
CPT = 80
EPAD = NW * CPT * CHUNK
NPAD = 10240
RPT = NPAD // NS

_f32 = jnp.float32


def _sc_mesh():
    return plsc.VectorSubcoreMesh(core_axis_name="c", subcore_axis_name="s")


def _sc_degree(dst2d, zvec):
    @functools.partial(
        pl.kernel,
        out_type=jax.ShapeDtypeStruct((NC * NPAD,), _f32),
        mesh=_sc_mesh(),
        scratch_types=[
            pltpu.VMEM((CPT, CHUNK), jnp.int32),
            pltpu.VMEM((CHUNK,), _f32),
            pltpu.VMEM_SHARED((NPAD,), _f32),
        ],
    )
    def k(dst_hbm, z_hbm, out_hbm, idx_v, ones_v, acc):
        c = lax.axis_index("c")
        s = lax.axis_index("s")
        wid = c * NS + s
        pltpu.sync_copy(dst_hbm.at[pl.ds(wid * CPT, CPT), :], idx_v)

        @pl.loop(0, CHUNK // 16)
        def _(i):
            ones_v[pl.ds(i * 16, 16)] = jnp.full((16,), 1.0, _f32)

        pltpu.sync_copy(z_hbm.at[pl.ds(s * RPT, RPT)], acc.at[pl.ds(s * RPT, RPT)])
        plsc.subcore_barrier()

        @pl.loop(0, CPT)
        def _(j):
            pltpu.sync_copy(ones_v, acc.at[idx_v.at[j]], add=True)

        plsc.subcore_barrier()
        pltpu.sync_copy(
            acc.at[pl.ds(s * RPT, RPT)],
            out_hbm.at[pl.ds(c * NPAD + s * RPT, RPT)],
        )

    return k(dst2d, zvec)


def _sc_layer(xn, src2d, dst2d, zrows):
    IB = 16
    NBLK = CPT // IB

    @functools.partial(
        pl.kernel,
        out_type=jax.ShapeDtypeStruct((NC * NPAD, D), _f32),
        mesh=_sc_mesh(),
        scratch_types=(
            [pltpu.VMEM((CPT, CHUNK), jnp.int32)]
            + [pltpu.VMEM((IB, CHUNK), jnp.int32)] * 2
            + [pltpu.VMEM((CHUNK, D), _f32)] * 2
            + [pltpu.VMEM_SHARED((NPAD, D), _f32)]
            + [pltpu.SemaphoreType.DMA] * 4
        ),
    )
    def k(xn_hbm, src_hbm, dst_hbm, z_hbm, out_hbm,
          si, di0, di1, r0, r1, acc, gs0, gs1, is0, is1):
        di = (di0, di1)
        rows = (r0, r1)
        gs = (gs0, gs1)
        isem = (is0, is1)
        c = lax.axis_index("c")
        s = lax.axis_index("s")
        wid = c * NS + s

        def dst_block(bi):
            return dst_hbm.at[pl.ds(wid * CPT + bi * IB, IB), :]

        pltpu.sync_copy(src_hbm.at[pl.ds(wid * CPT, CPT), :], si)
        pltpu.sync_copy(dst_block(0), di[0])
        pltpu.sync_copy(z_hbm, acc.at[pl.ds(s * RPT, RPT), :])
        plsc.subcore_barrier()

        for b in range(2):
            pltpu.async_copy(xn_hbm.at[si.at[b]], rows[b], gs[b])

        def pair_step(bi, p, t, last):
            for b in range(2):
                jl = 2 * t + b
                jg = bi * IB + jl
                pltpu.make_async_copy(xn_hbm.at[si.at[jg]], rows[b], gs[b]).wait()
                pltpu.sync_copy(rows[b], acc.at[di[p].at[jl]], add=True)
                if not last:
                    pltpu.async_copy(xn_hbm.at[si.at[jg + 2]], rows[b], gs[b])

        for bi in range(NBLK):
            p = bi % 2
            q = (bi + 1) % 2
            if bi > 0:
                pltpu.make_async_copy(dst_block(bi), di[p], isem[p]).wait()
            if bi + 1 < NBLK:
                pltpu.async_copy(dst_block(bi + 1), di[q], isem[q])

            if bi + 1 < NBLK:
                @pl.loop(0, IB // 2)
                def _(t):
                    pair_step(bi, p, t, last=False)
            else:
                @pl.loop(0, IB // 2 - 1)
                def _(t):
                    pair_step(bi, p, t, last=False)
                pair_step(bi, p, IB // 2 - 1, last=True)

        plsc.subcore_barrier()
        pltpu.sync_copy(
            acc.at[pl.ds(s * RPT, RPT), :],
            out_hbm.at[pl.ds(c * NPAD + s * RPT, RPT), :],
        )

    return k(xn, src2d, dst2d, zrows)


def _tc_prep(dpa, dpb, h):
    def body(dpa_ref, dpb_ref, h_ref, norm_ref, xn_ref):
        deg = dpa_ref[...] + dpb_ref[...]
        rid = lax.broadcasted_iota(jnp.int32, (NPAD, 1), 0)
        norm = jnp.where((deg > 0.0) & (rid < N), lax.rsqrt(deg), 0.0)
        norm_ref[...] = norm
        xn_ref[:N, :] = h_ref[...] * norm[:N]
        xn_ref[N:, :] = jnp.zeros((NPAD - N, D), _f32)

    return pl.pallas_call(
        body,
        out_shape=(
            jax.ShapeDtypeStruct((NPAD, 1), _f32),
            jax.ShapeDtypeStruct((NPAD, D), _f32),
        ),
    )(dpa, dpb, h)


_BR = 1280
_G = NPAD // _BR


def _tc_layer(pa, pb, norm, W, b):
    def body(pa_ref, pb_ref, norm_ref, w_ref, b_ref, xn_ref):
        agg = (pa_ref[...] + pb_ref[...]) * norm_ref[...]
        x = jnp.dot(agg, w_ref[...], preferred_element_type=_f32) + b_ref[...]
        x = x * jax.nn.sigmoid(x)
        xn_ref[...] = x * norm_ref[...]

    return pl.pallas_call(
        body,
        grid=(_G,),
        in_specs=[
            pl.BlockSpec((_BR, D), lambda i: (i, 0)),
            pl.BlockSpec((_BR, D), lambda i: (i, 0)),
            pl.BlockSpec((_BR, 1), lambda i: (i, 0)),
            pl.BlockSpec((D, D), lambda i: (0, 0)),
            pl.BlockSpec((1, D), lambda i: (0, 0)),
        ],
        out_specs=pl.BlockSpec((_BR, D), lambda i: (i, 0)),
        out_shape=jax.ShapeDtypeStruct((NPAD, D), _f32),
    )(pa, pb, norm, W, b)


def _tc_final(pa, pb, norm, W, b, PW0, PB0, PW1, PB1):
    def body(pa_ref, pb_ref, norm_ref, w_ref, b_ref, pw0, pb0, pw1, pb1,
             out_ref, acc_ref):
        i = pl.program_id(0)
        agg = (pa_ref[...] + pb_ref[...]) * norm_ref[...]
        x = jnp.dot(agg, w_ref[...], preferred_element_type=_f32) + b_ref[...]
        x = x * jax.nn.sigmoid(x)
        rid = i * _BR + lax.broadcasted_iota(jnp.int32, (_BR, 1), 0)
        x = jnp.where(rid < N, x, 0.0)

        @pl.when(i == 0)
        def _():
            acc_ref[...] = jnp.zeros((1, D), _f32)

        acc_ref[...] += jnp.sum(x, axis=0, keepdims=True)

        @pl.when(i == _G - 1)
        def _():
            m = acc_ref[...] * (1.0 / N)
            t = jnp.tanh(jnp.dot(m, pw0[...], preferred_element_type=_f32)
                         + pb0[...])
            out_ref[...] = (jnp.dot(t, pw1[...], preferred_element_type=_f32)
                            + pb1[...])

    return pl.pallas_call(
        body,
        grid=(_G,),
        in_specs=[
            pl.BlockSpec((_BR, D), lambda i: (i, 0)),
            pl.BlockSpec((_BR, D), lambda i: (i, 0)),
            pl.BlockSpec((_BR, 1), lambda i: (i, 0)),
            pl.BlockSpec((D, D), lambda i: (0, 0)),
            pl.BlockSpec((1, D), lambda i: (0, 0)),
            pl.BlockSpec((D, D), lambda i: (0, 0)),
            pl.BlockSpec((1, D), lambda i: (0, 0)),
            pl.BlockSpec((D, PW1.shape[1]), lambda i: (0, 0)),
            pl.BlockSpec((1, PW1.shape[1]), lambda i: (0, 0)),
        ],
        out_specs=pl.BlockSpec((1, PW1.shape[1]), lambda i: (0, 0)),
        out_shape=jax.ShapeDtypeStruct((1, PW1.shape[1]), _f32),
        scratch_shapes=[pltpu.VMEM((1, D), _f32)],
    )(pa, pb, norm, W, b, PW0, PB0, PW1, PB1)


def kernel(h, edge_index, Wg0, bg0, Wg1, bg1, Wg2, bg2, PW0, PB0, PW1, PB1):
    src = edge_index[0].astype(jnp.int32)
    dst = edge_index[1].astype(jnp.int32)
    pad = (N + jnp.arange(EPAD - E, dtype=jnp.int32) % (NPAD - N)).astype(jnp.int32)
    src2d = jnp.concatenate([src, pad]).reshape(EPAD // CHUNK, CHUNK)
    dst2d = jnp.concatenate([dst, pad]).reshape(EPAD // CHUNK, CHUNK)
    zrows = jnp.zeros((RPT, D), _f32)
    zvec = jnp.zeros((NPAD,), _f32)

    degp = _sc_degree(dst2d, zvec)
    dpa = degp[:NPAD].reshape(NPAD, 1)
    dpb = degp[NPAD:].reshape(NPAD, 1)
    norm, xn = _tc_prep(dpa, dpb, h)

    for (W, b) in ((Wg0, bg0), (Wg1, bg1)):
        pp = _sc_layer(xn, src2d, dst2d, zrows)
        xn = _tc_layer(pp[:NPAD], pp[NPAD:], norm, W, b.reshape(1, D))

    pp = _sc_layer(xn, src2d, dst2d, zrows)
    return _tc_final(
        pp[:NPAD], pp[NPAD:], norm, Wg2, bg2.reshape(1, D),
        PW0, PB0.reshape(1, -1), PW1, PB1.reshape(1, -1),
    )


# speedup vs baseline: 1.0143x; 1.0143x over previous
import functools

import jax
import jax.numpy as jnp
from jax import lax
from jax.experimental import pallas as pl
from jax.experimental.pallas import tpu as pltpu
from jax.experimental.pallas import tpu_sc as plsc

N = 10000
E = 320000
D = 128
NC, NS = 2, 16
NW = NC * NS
CHUNK = 128
CPT = 80
EPAD = NW * CPT * CHUNK
NPAD = 10240
RPT = NPAD // NS

_f32 = jnp.float32


def _sc_mesh():
    return plsc.VectorSubcoreMesh(core_axis_name="c", subcore_axis_name="s")


def _sc_degree(dst2d, zvec):
    @functools.partial(
        pl.kernel,
        out_type=jax.ShapeDtypeStruct((NC * NPAD,), _f32),
        mesh=_sc_mesh(),
        scratch_types=[
            pltpu.VMEM((CPT, CHUNK), jnp.int32),
            pltpu.VMEM((CHUNK,), _f32),
            pltpu.VMEM_SHARED((NPAD,), _f32),
        ],
    )
    def k(dst_hbm, z_hbm, out_hbm, idx_v, ones_v, acc):
        c = lax.axis_index("c")
        s = lax.axis_index("s")
        wid = c * NS + s
        pltpu.sync_copy(dst_hbm.at[pl.ds(wid * CPT, CPT), :], idx_v)

        @pl.loop(0, CHUNK // 16)
        def _(i):
            ones_v[pl.ds(i * 16, 16)] = jnp.full((16,), 1.0, _f32)

        pltpu.sync_copy(z_hbm.at[pl.ds(s * RPT, RPT)], acc.at[pl.ds(s * RPT, RPT)])
        plsc.subcore_barrier()

        @pl.loop(0, CPT)
        def _(j):
            pltpu.sync_copy(ones_v, acc.at[idx_v.at[j]], add=True)

        plsc.subcore_barrier()
        pltpu.sync_copy(
            acc.at[pl.ds(s * RPT, RPT)],
            out_hbm.at[pl.ds(c * NPAD + s * RPT, RPT)],
        )

    return k(dst2d, zvec)


def _sc_layer(xn, src2d, dst2d, zrows):
    IB = 16
    NBLK = CPT // IB

    @functools.partial(
        pl.kernel,
        out_type=jax.ShapeDtypeStruct((NC * NPAD, D), _f32),
        mesh=_sc_mesh(),
        scratch_types=(
            [pltpu.VMEM((CPT, CHUNK), jnp.int32)]
            + [pltpu.VMEM((IB, CHUNK), jnp.int32)] * 2
            + [pltpu.VMEM((CHUNK, D), _f32)] * 2
            + [pltpu.VMEM_SHARED((NPAD, D), _f32)]
            + [pltpu.SemaphoreType.DMA] * 4
        ),
    )
    def k(xn_hbm, src_hbm, dst_hbm, z_hbm, out_hbm,
          si, di0, di1, r0, r1, acc, gs0, gs1, is0, is1):
        di = (di0, di1)
        rows = (r0, r1)
        gs = (gs0, gs1)
        isem = (is0, is1)
        c = lax.axis_index("c")
        s = lax.axis_index("s")
        wid = c * NS + s

        def dst_block(bi):
            return dst_hbm.at[pl.ds(wid * CPT + bi * IB, IB), :]

        pltpu.sync_copy(src_hbm.at[pl.ds(wid * CPT, CPT), :], si)
        pltpu.sync_copy(dst_block(0), di[0])
        pltpu.sync_copy(z_hbm, acc.at[pl.ds(s * RPT, RPT), :])
        plsc.subcore_barrier()

        for b in range(2):
            pltpu.async_copy(xn_hbm.at[si.at[b]], rows[b], gs[b])

        def pair_step(bi, p, t, last):
            for b in range(2):
                jl = 2 * t + b
                jg = bi * IB + jl
                pltpu.make_async_copy(xn_hbm.at[si.at[jg]], rows[b], gs[b]).wait()
                pltpu.sync_copy(rows[b], acc.at[di[p].at[jl]], add=True)
                if not last:
                    pltpu.async_copy(xn_hbm.at[si.at[jg + 2]], rows[b], gs[b])

        for bi in range(NBLK):
            p = bi % 2
            q = (bi + 1) % 2
            if bi > 0:
                pltpu.make_async_copy(dst_block(bi), di[p], isem[p]).wait()
            if bi + 1 < NBLK:
                pltpu.async_copy(dst_block(bi + 1), di[q], isem[q])

            if bi + 1 < NBLK:
                @pl.loop(0, IB // 2)
                def _(t):
                    pair_step(bi, p, t, last=False)
            else:
                @pl.loop(0, IB // 2 - 1)
                def _(t):
                    pair_step(bi, p, t, last=False)
                pair_step(bi, p, IB // 2 - 1, last=True)

        plsc.subcore_barrier()
        pltpu.sync_copy(
            acc.at[pl.ds(s * RPT, RPT), :],
            out_hbm.at[pl.ds(c * NPAD + s * RPT, RPT), :],
        )

    return k(xn, src2d, dst2d, zrows)


def _tc_prep(dpa, dpb, h):
    def body(dpa_ref, dpb_ref, h_ref, norm_ref, xn_ref):
        deg = dpa_ref[...] + dpb_ref[...]
        rid = lax.broadcasted_iota(jnp.int32, (NPAD, 1), 0)
        norm = jnp.where((deg > 0.0) & (rid < N), lax.rsqrt(deg), 0.0)
        norm_ref[...] = norm
        xn_ref[:N, :] = h_ref[...] * norm[:N]
        xn_ref[N:, :] = jnp.zeros((NPAD - N, D), _f32)

    return pl.pallas_call(
        body,
        out_shape=(
            jax.ShapeDtypeStruct((NPAD, 1), _f32),
            jax.ShapeDtypeStruct((NPAD, D), _f32),
        ),
    )(dpa, dpb, h)


def _tc_layer(pa, pb, norm, W, b):
    def body(pa_ref, pb_ref, norm_ref, w_ref, b_ref, xn_ref):
        agg = (pa_ref[...] + pb_ref[...]) * norm_ref[...]
        x = jnp.dot(agg, w_ref[...], preferred_element_type=_f32) + b_ref[...]
        x = x * jax.nn.sigmoid(x)
        xn_ref[...] = x * norm_ref[...]

    return pl.pallas_call(
        body,
        out_shape=jax.ShapeDtypeStruct((NPAD, D), _f32),
    )(pa, pb, norm, W, b)


def _tc_final(pa, pb, norm, W, b, PW0, PB0, PW1, PB1):
    def body(pa_ref, pb_ref, norm_ref, w_ref, b_ref, pw0, pb0, pw1, pb1, out_ref):
        agg = (pa_ref[:N, :] + pb_ref[:N, :]) * norm_ref[:N, :]
        x = jnp.dot(agg, w_ref[...], preferred_element_type=_f32) + b_ref[...]
        x = x * jax.nn.sigmoid(x)
        m = jnp.mean(x, axis=0, keepdims=True)
        t = jnp.tanh(jnp.dot(m, pw0[...], preferred_element_type=_f32) + pb0[...])
        out_ref[...] = jnp.dot(t, pw1[...], preferred_element_type=_f32) + pb1[...]

    return pl.pallas_call(
        body,
        out_shape=jax.ShapeDtypeStruct((1, PW1.shape[1]), _f32),
    )(pa, pb, norm, W, b, PW0, PB0, PW1, PB1)


def kernel(h, edge_index, Wg0, bg0, Wg1, bg1, Wg2, bg2, PW0, PB0, PW1, PB1):
    src = edge_index[0].astype(jnp.int32)
    dst = edge_index[1].astype(jnp.int32)
    pad = (N + jnp.arange(EPAD - E, dtype=jnp.int32) % (NPAD - N)).astype(jnp.int32)
    src2d = jnp.concatenate([src, pad]).reshape(EPAD // CHUNK, CHUNK)
    dst2d = jnp.concatenate([dst, pad]).reshape(EPAD // CHUNK, CHUNK)
    zrows = jnp.zeros((RPT, D), _f32)
    zvec = jnp.zeros((NPAD,), _f32)

    degp = _sc_degree(dst2d, zvec)
    dpa = degp[:NPAD].reshape(NPAD, 1)
    dpb = degp[NPAD:].reshape(NPAD, 1)
    norm, xn = _tc_prep(dpa, dpb, h)

    for (W, b) in ((Wg0, bg0), (Wg1, bg1)):
        pp = _sc_layer(xn, src2d, dst2d, zrows)
        xn = _tc_layer(pp[:NPAD], pp[NPAD:], norm, W, b.reshape(1, D))

    pp = _sc_layer(xn, src2d, dst2d, zrows)
    return _tc_final(
        pp[:NPAD], pp[NPAD:], norm, Wg2, bg2.reshape(1, D),
        PW0, PB0.reshape(1, -1), PW1, PB1.reshape(1, -1),
    )


# speedup vs baseline: 1.0274x; 1.0129x over previous
import functools

import jax
import jax.numpy as jnp
from jax import lax
from jax.experimental import pallas as pl
from jax.experimental.pallas import tpu as pltpu
from jax.experimental.pallas import tpu_sc as plsc

N = 10000
E = 320000
D = 128
NC, NS = 2, 16
NW = NC * NS
CHUNK = 128
CPT = 80
EPAD = NW * CPT * CHUNK
NPAD = 10240
RPT = NPAD // NS

_f32 = jnp.float32


def _sc_mesh():
    return plsc.VectorSubcoreMesh(core_axis_name="c", subcore_axis_name="s")


def _sc_degree(dst2d, zvec):
    @functools.partial(
        pl.kernel,
        out_type=jax.ShapeDtypeStruct((NC * NPAD,), _f32),
        mesh=_sc_mesh(),
        scratch_types=[
            pltpu.VMEM((CPT, CHUNK), jnp.int32),
            pltpu.VMEM((CHUNK,), _f32),
            pltpu.VMEM_SHARED((NPAD,), _f32),
        ],
    )
    def k(dst_hbm, z_hbm, out_hbm, idx_v, ones_v, acc):
        c = lax.axis_index("c")
        s = lax.axis_index("s")
        wid = c * NS + s
        pltpu.sync_copy(dst_hbm.at[pl.ds(wid * CPT, CPT), :], idx_v)

        @pl.loop(0, CHUNK // 16)
        def _(i):
            ones_v[pl.ds(i * 16, 16)] = jnp.full((16,), 1.0, _f32)

        pltpu.sync_copy(z_hbm.at[pl.ds(s * RPT, RPT)], acc.at[pl.ds(s * RPT, RPT)])
        plsc.subcore_barrier()

        @pl.loop(0, CPT)
        def _(j):
            pltpu.sync_copy(ones_v, acc.at[idx_v.at[j]], add=True)

        plsc.subcore_barrier()
        pltpu.sync_copy(
            acc.at[pl.ds(s * RPT, RPT)],
            out_hbm.at[pl.ds(c * NPAD + s * RPT, RPT)],
        )

    return k(dst2d, zvec)


def _sc_layer(xn, src2d, dst2d, zrows):
    IB = 16
    NBLK = CPT // IB

    @functools.partial(
        pl.kernel,
        out_type=jax.ShapeDtypeStruct((NC * NPAD, D), _f32),
        mesh=_sc_mesh(),
        scratch_types=(
            [pltpu.VMEM((CPT, CHUNK), jnp.int32)]
            + [pltpu.VMEM((IB, CHUNK), jnp.int32)] * 2
            + [pltpu.VMEM((CHUNK, D), _f32)] * 2
            + [pltpu.VMEM_SHARED((NPAD, D), _f32)]
            + [pltpu.SemaphoreType.DMA] * 5
        ),
    )
    def k(xn_hbm, src_hbm, dst_hbm, z_hbm, out_hbm,
          si, di0, di1, r0, r1, acc, gs0, gs1, is0, is1, zs):
        di = (di0, di1)
        rows = (r0, r1)
        gs = (gs0, gs1)
        isem = (is0, is1)
        c = lax.axis_index("c")
        s = lax.axis_index("s")
        wid = c * NS + s

        def dst_block(bi):
            return dst_hbm.at[pl.ds(wid * CPT + bi * IB, IB), :]

        pltpu.async_copy(z_hbm, acc.at[pl.ds(s * RPT, RPT), :], zs)
        pltpu.sync_copy(src_hbm.at[pl.ds(wid * CPT, CPT), :], si)
        pltpu.sync_copy(dst_block(0), di[0])
        for b in range(2):
            pltpu.async_copy(xn_hbm.at[si.at[b]], rows[b], gs[b])
        pltpu.make_async_copy(z_hbm, acc.at[pl.ds(s * RPT, RPT), :], zs).wait()
        plsc.subcore_barrier()

        def pair_step(bi, p, t, last):
            for b in range(2):
                jl = 2 * t + b
                jg = bi * IB + jl
                pltpu.make_async_copy(xn_hbm.at[si.at[jg]], rows[b], gs[b]).wait()
                pltpu.sync_copy(rows[b], acc.at[di[p].at[jl]], add=True)
                if not last:
                    pltpu.async_copy(xn_hbm.at[si.at[jg + 2]], rows[b], gs[b])

        for bi in range(NBLK):
            p = bi % 2
            q = (bi + 1) % 2
            if bi > 0:
                pltpu.make_async_copy(dst_block(bi), di[p], isem[p]).wait()
            if bi + 1 < NBLK:
                pltpu.async_copy(dst_block(bi + 1), di[q], isem[q])

            if bi + 1 < NBLK:
                @pl.loop(0, IB // 2)
                def _(t):
                    pair_step(bi, p, t, last=False)
            else:
                @pl.loop(0, IB // 2 - 1)
                def _(t):
                    pair_step(bi, p, t, last=False)
                pair_step(bi, p, IB // 2 - 1, last=True)

        plsc.subcore_barrier()
        pltpu.sync_copy(
            acc.at[pl.ds(s * RPT, RPT), :],
            out_hbm.at[pl.ds(c * NPAD + s * RPT, RPT), :],
        )

    return k(xn, src2d, dst2d, zrows)


def _tc_prep(dpa, dpb, h):
    def body(dpa_ref, dpb_ref, h_ref, norm_ref, xn_ref):
        deg = dpa_ref[...] + dpb_ref[...]
        rid = lax.broadcasted_iota(jnp.int32, (NPAD, 1), 0)
        norm = jnp.where((deg > 0.0) & (rid < N), lax.rsqrt(deg), 0.0)
        norm_ref[...] = norm
        xn_ref[:N, :] = h_ref[...] * norm[:N]
        xn_ref[N:, :] = jnp.zeros((NPAD - N, D), _f32)

    return pl.pallas_call(
        body,
        out_shape=(
            jax.ShapeDtypeStruct((NPAD, 1), _f32),
            jax.ShapeDtypeStruct((NPAD, D), _f32),
        ),
    )(dpa, dpb, h)


def _tc_layer(pa, pb, norm, W, b):
    def body(pa_ref, pb_ref, norm_ref, w_ref, b_ref, xn_ref):
        agg = (pa_ref[...] + pb_ref[...]) * norm_ref[...]
        x = jnp.dot(agg, w_ref[...], preferred_element_type=_f32) + b_ref[...]
        x = x * jax.nn.sigmoid(x)
        xn_ref[...] = x * norm_ref[...]

    return pl.pallas_call(
        body,
        out_shape=jax.ShapeDtypeStruct((NPAD, D), _f32),
    )(pa, pb, norm, W, b)


def _tc_final(pa, pb, norm, W, b, PW0, PB0, PW1, PB1):
    def body(pa_ref, pb_ref, norm_ref, w_ref, b_ref, pw0, pb0, pw1, pb1, out_ref):
        agg = (pa_ref[:N, :] + pb_ref[:N, :]) * norm_ref[:N, :]
        x = jnp.dot(agg, w_ref[...], preferred_element_type=_f32) + b_ref[...]
        x = x * jax.nn.sigmoid(x)
        m = jnp.mean(x, axis=0, keepdims=True)
        t = jnp.tanh(jnp.dot(m, pw0[...], preferred_element_type=_f32) + pb0[...])
        out_ref[...] = jnp.dot(t, pw1[...], preferred_element_type=_f32) + pb1[...]

    return pl.pallas_call(
        body,
        out_shape=jax.ShapeDtypeStruct((1, PW1.shape[1]), _f32),
    )(pa, pb, norm, W, b, PW0, PB0, PW1, PB1)


def kernel(h, edge_index, Wg0, bg0, Wg1, bg1, Wg2, bg2, PW0, PB0, PW1, PB1):
    src = edge_index[0].astype(jnp.int32)
    dst = edge_index[1].astype(jnp.int32)
    pad = (N + jnp.arange(EPAD - E, dtype=jnp.int32) % (NPAD - N)).astype(jnp.int32)
    src2d = jnp.concatenate([src, pad]).reshape(EPAD // CHUNK, CHUNK)
    dst2d = jnp.concatenate([dst, pad]).reshape(EPAD // CHUNK, CHUNK)
    zrows = jnp.zeros((RPT, D), _f32)
    zvec = jnp.zeros((NPAD,), _f32)

    degp = _sc_degree(dst2d, zvec)
    dpa = degp[:NPAD].reshape(NPAD, 1)
    dpb = degp[NPAD:].reshape(NPAD, 1)
    norm, xn = _tc_prep(dpa, dpb, h)

    for (W, b) in ((Wg0, bg0), (Wg1, bg1)):
        pp = _sc_layer(xn, src2d, dst2d, zrows)
        xn = _tc_layer(pp[:NPAD], pp[NPAD:], norm, W, b.reshape(1, D))

    pp = _sc_layer(xn, src2d, dst2d, zrows)
    return _tc_final(
        pp[:NPAD], pp[NPAD:], norm, Wg2, bg2.reshape(1, D),
        PW0, PB0.reshape(1, -1), PW1, PB1.reshape(1, -1),
    )


# speedup vs baseline: 1.0317x; 1.0042x over previous
import functools

import jax
import jax.numpy as jnp
from jax import lax
from jax.experimental import pallas as pl
from jax.experimental.pallas import tpu as pltpu
from jax.experimental.pallas import tpu_sc as plsc

N = 10000
E = 320000
D = 128
NC, NS = 2, 16
NW = NC * NS
CHUNK = 128
CPT = 80
EPAD = NW * CPT * CHUNK
NPAD = 10240
RPT = NPAD // NS

_f32 = jnp.float32


def _sc_mesh():
    return plsc.VectorSubcoreMesh(core_axis_name="c", subcore_axis_name="s")


def _sc_degree(dst2d, zvec):
    @functools.partial(
        pl.kernel,
        out_type=jax.ShapeDtypeStruct((NC * NPAD,), _f32),
        mesh=_sc_mesh(),
        scratch_types=[
            pltpu.VMEM((CPT, CHUNK), jnp.int32),
            pltpu.VMEM((CHUNK,), _f32),
            pltpu.VMEM_SHARED((NPAD,), _f32),
            pltpu.SemaphoreType.DMA,
        ],
    )
    def k(dst_hbm, z_hbm, out_hbm, idx_v, ones_v, acc, zs):
        c = lax.axis_index("c")
        s = lax.axis_index("s")
        wid = c * NS + s
        pltpu.async_copy(
            z_hbm.at[pl.ds(s * RPT, RPT)], acc.at[pl.ds(s * RPT, RPT)], zs)
        pltpu.sync_copy(dst_hbm.at[pl.ds(wid * CPT, CPT), :], idx_v)

        @pl.loop(0, CHUNK // 16)
        def _(i):
            ones_v[pl.ds(i * 16, 16)] = jnp.full((16,), 1.0, _f32)

        pltpu.make_async_copy(
            z_hbm.at[pl.ds(s * RPT, RPT)], acc.at[pl.ds(s * RPT, RPT)], zs).wait()
        plsc.subcore_barrier()

        @pl.loop(0, CPT)
        def _(j):
            pltpu.sync_copy(ones_v, acc.at[idx_v.at[j]], add=True)

        plsc.subcore_barrier()
        pltpu.sync_copy(
            acc.at[pl.ds(s * RPT, RPT)],
            out_hbm.at[pl.ds(c * NPAD + s * RPT, RPT)],
        )

    return k(dst2d, zvec)


def _sc_layer(xn, src2d, dst2d, zrows):
    IB = 16
    NBLK = CPT // IB

    @functools.partial(
        pl.kernel,
        out_type=jax.ShapeDtypeStruct((NC * NPAD, D), _f32),
        mesh=_sc_mesh(),
        scratch_types=(
            [pltpu.VMEM((CPT, CHUNK), jnp.int32)]
            + [pltpu.VMEM((IB, CHUNK), jnp.int32)] * 2
            + [pltpu.VMEM((CHUNK, D), _f32)] * 2
            + [pltpu.VMEM_SHARED((NPAD, D), _f32)]
            + [pltpu.SemaphoreType.DMA] * 5
        ),
    )
    def k(xn_hbm, src_hbm, dst_hbm, z_hbm, out_hbm,
          si, di0, di1, r0, r1, acc, gs0, gs1, is0, is1, zs):
        di = (di0, di1)
        rows = (r0, r1)
        gs = (gs0, gs1)
        isem = (is0, is1)
        c = lax.axis_index("c")
        s = lax.axis_index("s")
        wid = c * NS + s

        def dst_block(bi):
            return dst_hbm.at[pl.ds(wid * CPT + bi * IB, IB), :]

        pltpu.async_copy(z_hbm, acc.at[pl.ds(s * RPT, RPT), :], zs)
        pltpu.sync_copy(src_hbm.at[pl.ds(wid * CPT, CPT), :], si)
        pltpu.sync_copy(dst_block(0), di[0])
        for b in range(2):
            pltpu.async_copy(xn_hbm.at[si.at[b]], rows[b], gs[b])
        pltpu.make_async_copy(z_hbm, acc.at[pl.ds(s * RPT, RPT), :], zs).wait()
        plsc.subcore_barrier()

        def pair_step(bi, p, t, last):
            for b in range(2):
                jl = 2 * t + b
                jg = bi * IB + jl
                pltpu.make_async_copy(xn_hbm.at[si.at[jg]], rows[b], gs[b]).wait()
                pltpu.sync_copy(rows[b], acc.at[di[p].at[jl]], add=True)
                if not last:
                    pltpu.async_copy(xn_hbm.at[si.at[jg + 2]], rows[b], gs[b])

        for bi in range(NBLK):
            p = bi % 2
            q = (bi + 1) % 2
            if bi > 0:
                pltpu.make_async_copy(dst_block(bi), di[p], isem[p]).wait()
            if bi + 1 < NBLK:
                pltpu.async_copy(dst_block(bi + 1), di[q], isem[q])

            if bi + 1 < NBLK:
                @pl.loop(0, IB // 2)
                def _(t):
                    pair_step(bi, p, t, last=False)
            else:
                @pl.loop(0, IB // 2 - 1)
                def _(t):
                    pair_step(bi, p, t, last=False)
                pair_step(bi, p, IB // 2 - 1, last=True)

        plsc.subcore_barrier()
        pltpu.sync_copy(
            acc.at[pl.ds(s * RPT, RPT), :],
            out_hbm.at[pl.ds(c * NPAD + s * RPT, RPT), :],
        )

    return k(xn, src2d, dst2d, zrows)


def _tc_prep(dpa, dpb, h):
    def body(dpa_ref, dpb_ref, h_ref, norm_ref, xn_ref):
        deg = dpa_ref[...] + dpb_ref[...]
        rid = lax.broadcasted_iota(jnp.int32, (NPAD, 1), 0)
        norm = jnp.where((deg > 0.0) & (rid < N), lax.rsqrt(deg), 0.0)
        norm_ref[...] = norm
        xn_ref[:N, :] = h_ref[...] * norm[:N]
        xn_ref[N:, :] = jnp.zeros((NPAD - N, D), _f32)

    return pl.pallas_call(
        body,
        out_shape=(
            jax.ShapeDtypeStruct((NPAD, 1), _f32),
            jax.ShapeDtypeStruct((NPAD, D), _f32),
        ),
    )(dpa, dpb, h)


def _tc_layer(pa, pb, norm, W, b):
    def body(pa_ref, pb_ref, norm_ref, w_ref, b_ref, xn_ref):
        agg = (pa_ref[...] + pb_ref[...]) * norm_ref[...]
        x = jnp.dot(agg, w_ref[...], preferred_element_type=_f32) + b_ref[...]
        x = x * jax.nn.sigmoid(x)
        xn_ref[...] = x * norm_ref[...]

    return pl.pallas_call(
        body,
        out_shape=jax.ShapeDtypeStruct((NPAD, D), _f32),
    )(pa, pb, norm, W, b)


def _tc_final(pa, pb, norm, W, b, PW0, PB0, PW1, PB1):
    def body(pa_ref, pb_ref, norm_ref, w_ref, b_ref, pw0, pb0, pw1, pb1, out_ref):
        agg = (pa_ref[:N, :] + pb_ref[:N, :]) * norm_ref[:N, :]
        x = jnp.dot(agg, w_ref[...], preferred_element_type=_f32) + b_ref[...]
        x = x * jax.nn.sigmoid(x)
        m = jnp.mean(x, axis=0, keepdims=True)
        t = jnp.tanh(jnp.dot(m, pw0[...], preferred_element_type=_f32) + pb0[...])
        out_ref[...] = jnp.dot(t, pw1[...], preferred_element_type=_f32) + pb1[...]

    return pl.pallas_call(
        body,
        out_shape=jax.ShapeDtypeStruct((1, PW1.shape[1]), _f32),
    )(pa, pb, norm, W, b, PW0, PB0, PW1, PB1)


def kernel(h, edge_index, Wg0, bg0, Wg1, bg1, Wg2, bg2, PW0, PB0, PW1, PB1):
    src = edge_index[0].astype(jnp.int32)
    dst = edge_index[1].astype(jnp.int32)
    pad = (N + jnp.arange(EPAD - E, dtype=jnp.int32) % (NPAD - N)).astype(jnp.int32)
    src2d = jnp.concatenate([src, pad]).reshape(EPAD // CHUNK, CHUNK)
    dst2d = jnp.concatenate([dst, pad]).reshape(EPAD // CHUNK, CHUNK)
    zrows = jnp.zeros((RPT, D), _f32)
    zvec = jnp.zeros((NPAD,), _f32)

    degp = _sc_degree(dst2d, zvec)
    dpa = degp[:NPAD].reshape(NPAD, 1)
    dpb = degp[NPAD:].reshape(NPAD, 1)
    norm, xn = _tc_prep(dpa, dpb, h)

    for (W, b) in ((Wg0, bg0), (Wg1, bg1)):
        pp = _sc_layer(xn, src2d, dst2d, zrows)
        xn = _tc_layer(pp[:NPAD], pp[NPAD:], norm, W, b.reshape(1, D))

    pp = _sc_layer(xn, src2d, dst2d, zrows)
    return _tc_final(
        pp[:NPAD], pp[NPAD:], norm, Wg2, bg2.reshape(1, D),
        PW0, PB0.reshape(1, -1), PW1, PB1.reshape(1, -1),
    )
